# baseline (speedup 1.0000x reference)
import functools

import numpy as np
import jax
import jax.numpy as jnp
from jax import lax
from jax.experimental import pallas as pl
from jax.experimental.pallas import tpu as pltpu
from jax.experimental.pallas import tpu_sc as plsc


def _crop_consts(N, T, temporal_unit=0, seed=0):
    rng = np.random.RandomState(seed)
    crop_l = int(rng.randint(2 ** (temporal_unit + 1), T + 1))
    crop_left = int(rng.randint(T - crop_l + 1))
    crop_right = crop_left + crop_l
    crop_eleft = int(rng.randint(crop_left + 1))
    crop_eright = int(rng.randint(crop_right, T + 1))
    crop_offset = rng.randint(-crop_eleft, T - crop_eright + 1, size=N)
    starts = [int(s) for s in (crop_offset + crop_left)]
    return crop_l, starts


@functools.partial(jax.jit, static_argnums=(1, 2))
def _run(x, crop_l, starts):
    N, T, D = x.shape
    mesh = plsc.VectorSubcoreMesh(core_axis_name="c", subcore_axis_name="s")
    info = plsc.get_sparse_core_info()
    NC, NS = info.num_cores, info.num_subcores
    NW = NC * NS
    rows_per_w = N // NW

    row_elems = crop_l * D

    @functools.partial(
        pl.kernel,
        out_type=jax.ShapeDtypeStruct((N * crop_l * D,), jnp.float32),
        scratch_types=[pltpu.SemaphoreType.DMA],
        mesh=mesh,
    )
    def k(x_hbm, out_hbm, sem):
        wid = lax.axis_index("s") * NC + lax.axis_index("c")
        for w in range(NW):
            @pl.when(wid == w)
            def _():
                copies = []
                for j in range(rows_per_w):
                    r = w * rows_per_w + j
                    copies.append(pltpu.make_async_copy(
                        x_hbm.at[pl.ds((r * T + starts[r]) * D, row_elems)],
                        out_hbm.at[pl.ds(r * row_elems, row_elems)], sem))
                for cp in copies:
                    cp.start()
                for cp in copies:
                    cp.wait()

    return k(x.reshape(N * T * D)).reshape(N, crop_l, D)


def kernel(x):
    N, T, D = x.shape
    crop_l, starts = _crop_consts(N, T)
    out = _run(x, crop_l, tuple(starts))
    return (out, out)


# speedup vs baseline: 10.9369x; 10.9369x over previous
import functools

import numpy as np
import jax
import jax.numpy as jnp
from jax import lax
from jax.experimental import pallas as pl
from jax.experimental.pallas import tpu as pltpu
from jax.experimental.pallas import tpu_sc as plsc


def _crop_consts(N, T, temporal_unit=0, seed=0):
    rng = np.random.RandomState(seed)
    crop_l = int(rng.randint(2 ** (temporal_unit + 1), T + 1))
    crop_left = int(rng.randint(T - crop_l + 1))
    crop_right = crop_left + crop_l
    crop_eleft = int(rng.randint(crop_left + 1))
    crop_eright = int(rng.randint(crop_right, T + 1))
    crop_offset = rng.randint(-crop_eleft, T - crop_eright + 1, size=N)
    starts = (crop_offset + crop_left).astype(np.int64)
    return crop_l, starts


_CH = 128


@functools.partial(jax.jit, static_argnums=(2, 3, 4))
def _run(x2d, idx, N, crop_l, D):
    n_chunks = idx.shape[1]
    rem = crop_l - (n_chunks - 1) * _CH

    mesh = plsc.VectorSubcoreMesh(core_axis_name="c", subcore_axis_name="s")
    info = plsc.get_sparse_core_info()
    NC, NS = info.num_cores, info.num_subcores
    NW = NC * NS
    rows_per_w = N // NW
    n_q = rows_per_w * n_chunks

    @functools.partial(
        pl.kernel,
        out_type=jax.ShapeDtypeStruct((N, crop_l, D), jnp.float32),
        scratch_types=[
            pltpu.VMEM((rows_per_w, n_chunks, _CH), jnp.int32),
            pltpu.VMEM((2, _CH, D), jnp.float32),
            pltpu.SemaphoreType.DMA,
            pltpu.SemaphoreType.DMA,
        ],
        mesh=mesh,
    )
    def k(x_hbm, idx_hbm, out_hbm, idx_v, buf_v, sem0, sem1):
        wid = lax.axis_index("s") * NC + lax.axis_index("c")
        sems = (sem0, sem1)
        for j in range(rows_per_w):
            pltpu.sync_copy(idx_hbm.at[wid * rows_per_w + j], idx_v.at[j])

        def gather(q):
            b = q % 2
            return pltpu.make_async_copy(
                x_hbm.at[idx_v.at[q // n_chunks, q % n_chunks]],
                buf_v.at[b], sems[b])

        gather(0).start()
        for q in range(n_q):
            b = q % 2
            if q + 1 < n_q:
                gather(q + 1).start()
            gather(q).wait()
            c = q % n_chunks
            L = _CH if c < n_chunks - 1 else rem
            r = wid * rows_per_w + q // n_chunks
            pltpu.sync_copy(buf_v.at[b, pl.ds(0, L)],
                            out_hbm.at[r, pl.ds(c * _CH, L)])

    return k(x2d, idx)


def kernel(x):
    N, T, D = x.shape
    crop_l, starts = _crop_consts(N, T)
    n_chunks = (crop_l + _CH - 1) // _CH
    t = np.minimum(np.arange(n_chunks * _CH, dtype=np.int64), crop_l - 1)
    idx = (np.arange(N, dtype=np.int64)[:, None] * T
           + starts[:, None] + t[None, :]).astype(np.int32)
    idx = jnp.asarray(idx.reshape(N, n_chunks, _CH))
    out = _run(x.reshape(N * T, D), idx, N, crop_l, D)
    return (out, out)
